# baseline (speedup 1.0000x reference)
import functools

import jax
import jax.numpy as jnp
from jax import lax
from jax.experimental import pallas as pl
from jax.experimental.pallas import tpu as pltpu
from jax.experimental.pallas import tpu_sc as plsc

F32 = jnp.float32


def _swish(x):
    return x * (0.5 + 0.5 * jnp.tanh(0.5 * x))


def _node_pre_body(nf_ref, v16_ref, At_ref, Bt_ref, be1_ref,
                   Wv1t_ref, bv1_ref, wv2_ref, bv2_ref,
                   p_ref, q_ref, vel16_ref):
    bn = nf_ref.shape[0]
    X = nf_ref[...].reshape(bn * 3, 128)
    p_ref[...] = (jnp.dot(X, At_ref[...], preferred_element_type=F32)
                  + be1_ref[...]).reshape(bn, 3, 128)
    q_ref[...] = jnp.dot(X, Bt_ref[...],
                         preferred_element_type=F32).reshape(bn, 3, 128)
    V = _swish(jnp.dot(X, Wv1t_ref[...], preferred_element_type=F32)
               + bv1_ref[...])
    sv = jnp.sum(V * wv2_ref[...], axis=1).reshape(bn, 3) + bv2_ref[0, 0]
    sv16 = jnp.concatenate([sv, jnp.zeros((bn, 13), F32)], axis=1)
    vel16_ref[...] = v16_ref[...] * sv16


def _node_pre(nf, v16, At, Bt, be1, Wv1t, bv1, wv2, bv2):
    n = nf.shape[0]
    bn = 400
    wspec = lambda s: pl.BlockSpec(s, lambda i: (0,) * len(s))
    return pl.pallas_call(
        _node_pre_body,
        grid=(n // bn,),
        in_specs=[
            pl.BlockSpec((bn, 3, 128), lambda i: (i, 0, 0)),
            pl.BlockSpec((bn, 16), lambda i: (i, 0)),
            wspec((128, 128)), wspec((128, 128)), wspec((1, 128)),
            wspec((128, 128)), wspec((1, 128)), wspec((1, 128)),
            wspec((1, 1)),
        ],
        out_specs=[
            pl.BlockSpec((bn, 3, 128), lambda i: (i, 0, 0)),
            pl.BlockSpec((bn, 3, 128), lambda i: (i, 0, 0)),
            pl.BlockSpec((bn, 16), lambda i: (i, 0)),
        ],
        out_shape=[
            jax.ShapeDtypeStruct((n, 3, 128), F32),
            jax.ShapeDtypeStruct((n, 3, 128), F32),
            jax.ShapeDtypeStruct((n, 16), F32),
        ],
    )(nf, v16, At, Bt, be1, Wv1t, bv1, wv2, bv2)


def _sc_gather(P3, Q3, c128, tgt_g, src_g):
    eg = tgt_g.shape[0]
    chunk = eg // 16
    eb = 80
    nblk = chunk // eb
    mesh = plsc.VectorSubcoreMesh(core_axis_name="c", subcore_axis_name="s")

    @functools.partial(
        pl.kernel,
        out_type=[
            jax.ShapeDtypeStruct((eg, 3, 128), F32),
            jax.ShapeDtypeStruct((eg, 3, 128), F32),
            jax.ShapeDtypeStruct((eg, 128), F32),
            jax.ShapeDtypeStruct((eg, 128), F32),
        ],
        mesh=mesh,
        scratch_types=[
            pltpu.VMEM((chunk,), jnp.int32),
            pltpu.VMEM((eb, 3, 128), F32),
            pltpu.VMEM((eb, 3, 128), F32),
            pltpu.VMEM((eb, 128), F32),
            pltpu.VMEM((eb, 128), F32),
            pltpu.SemaphoreType.DMA,
            pltpu.SemaphoreType.DMA,
            pltpu.SemaphoreType.DMA,
            pltpu.SemaphoreType.DMA,
        ],
    )
    def k(P_h, Q_h, c_h, tgt_h, src_h, gtf_h, gsf_h, ct_h, cs_h,
          idxa, fbuf0, fbuf1, cbuf0, cbuf1, semf0, semf1, semc0, semc1):
        cid = lax.axis_index("c")
        tid = lax.axis_index("s")
        base = tid * chunk

        def make(feat_src, feat_h, coord_h, idx_h):
            def _():
                pltpu.sync_copy(idx_h.at[pl.ds(base, chunk)], idxa)

                def pair(j2, _):
                    o0 = 2 * j2 * eb
                    o1 = o0 + eb
                    f0 = pltpu.async_copy(
                        feat_src.at[idxa.at[pl.ds(o0, eb)]], fbuf0, semf0)
                    c0 = pltpu.async_copy(
                        c_h.at[idxa.at[pl.ds(o0, eb)]], cbuf0, semc0)
                    f1 = pltpu.async_copy(
                        feat_src.at[idxa.at[pl.ds(o1, eb)]], fbuf1, semf1)
                    c1 = pltpu.async_copy(
                        c_h.at[idxa.at[pl.ds(o1, eb)]], cbuf1, semc1)
                    f0.wait()
                    c0.wait()
                    d0 = pl.multiple_of(base + o0, 8)
                    pltpu.sync_copy(fbuf0, feat_h.at[pl.ds(d0, eb)])
                    pltpu.sync_copy(cbuf0, coord_h.at[pl.ds(d0, eb)])
                    f1.wait()
                    c1.wait()
                    d1 = pl.multiple_of(base + o1, 8)
                    pltpu.sync_copy(fbuf1, feat_h.at[pl.ds(d1, eb)])
                    pltpu.sync_copy(cbuf1, coord_h.at[pl.ds(d1, eb)])
                    return 0

                lax.fori_loop(0, nblk // 2, pair, 0)

            return _

        pl.when(cid == 0)(make(P_h, gtf_h, ct_h, tgt_h))
        pl.when(cid == 1)(make(Q_h, gsf_h, cs_h, src_h))

    return k(P3, Q3, c128, tgt_g, src_g)


def _edge_mlp_body(gtf_ref, gsf_ref, ct_ref, cs_ref, w1_ref, We2t_ref,
                   be2_ref, Wc1t_ref, bc1_ref, ones_ref, Ws_ref, bc2_ref,
                   msg_ref, cm_ref, *, valid_rows):
    be = gtf_ref.shape[0]
    relc = ct_ref[...] - cs_ref[...]
    distb = jnp.dot(relc * relc, ones_ref[...], preferred_element_type=F32)
    X0 = (gtf_ref[...] + gsf_ref[...]
          + distb[:, None, :] * w1_ref[...][None])
    X = _swish(X0).reshape(be * 3, 128)
    Y = _swish(jnp.dot(X, We2t_ref[...], preferred_element_type=F32)
               + be2_ref[...])
    m = jnp.where(pl.program_id(0) * be < valid_rows, 1.0, 0.0).astype(F32)
    msg_ref[...] = Y.reshape(be, 3, 128) * m
    C = _swish(jnp.dot(Y, Wc1t_ref[...], preferred_element_type=F32)
               + bc1_ref[...]).reshape(be, 3, 128)
    s128 = (jnp.dot(C[:, 0, :], Ws_ref[0:128, :],
                    preferred_element_type=F32)
            + jnp.dot(C[:, 1, :], Ws_ref[128:256, :],
                      preferred_element_type=F32)
            + jnp.dot(C[:, 2, :], Ws_ref[256:384, :],
                      preferred_element_type=F32))
    cm_ref[...] = (relc * (s128 + bc2_ref[0, 0]) * m)[:, None, :]


def _edge_mlp(gtf, gsf, ct, cs, w1, We2t, be2, Wc1t, bc1, ones128, Ws, bc2,
              valid_rows):
    eg = gtf.shape[0]
    be = 640
    wspec = lambda s: pl.BlockSpec(s, lambda i: (0,) * len(s))
    return pl.pallas_call(
        functools.partial(_edge_mlp_body, valid_rows=valid_rows),
        grid=(eg // be,),
        in_specs=[
            pl.BlockSpec((be, 3, 128), lambda i: (i, 0, 0)),
            pl.BlockSpec((be, 3, 128), lambda i: (i, 0, 0)),
            pl.BlockSpec((be, 128), lambda i: (i, 0)),
            pl.BlockSpec((be, 128), lambda i: (i, 0)),
            wspec((1, 128)), wspec((128, 128)), wspec((1, 128)),
            wspec((128, 128)), wspec((1, 128)), wspec((128, 128)),
            wspec((384, 128)), wspec((1, 1)),
        ],
        out_specs=[
            pl.BlockSpec((be, 3, 128), lambda i: (i, 0, 0)),
            pl.BlockSpec((be, 1, 128), lambda i: (i, 0, 0)),
        ],
        out_shape=[
            jax.ShapeDtypeStruct((eg, 3, 128), F32),
            jax.ShapeDtypeStruct((eg, 1, 128), F32),
        ],
    )(gtf, gsf, ct, cs, w1, We2t, be2, Wc1t, bc1, ones128, Ws, bc2)


def _sc_scatter(MSG, CM, tgt, n):
    e = tgt.shape[0]
    nsub = 16
    chunk = e // nsub
    eb = 80
    nblk = chunk // eb
    nrow = n // nsub
    zb = 25
    nz = nrow // zb
    mesh = plsc.VectorSubcoreMesh(core_axis_name="c", subcore_axis_name="s")

    @functools.partial(
        pl.kernel,
        out_type=[
            jax.ShapeDtypeStruct((n, 3, 128), F32),
            jax.ShapeDtypeStruct((n, 1, 128), F32),
        ],
        mesh=mesh,
        scratch_types=[
            pltpu.VMEM_SHARED((n, 1, 128), F32),
            pltpu.VMEM((chunk,), jnp.int32),
            pltpu.VMEM((eb, 1, 128), F32),
            pltpu.VMEM((eb, 1, 128), F32),
            pltpu.VMEM((zb, 1, 128), F32),
            pltpu.SemaphoreType.DMA,
            pltpu.SemaphoreType.DMA,
        ],
    )
    def k(MSG_h, CM_h, tgt_h, A_h, CMA_h, acc, idxa, dbuf0, dbuf1, zbuf,
          sem0, sem1):
        cid = lax.axis_index("c")
        tid = lax.axis_index("s")
        base = tid * chunk

        def zrow(r, _):
            for kk in range(8):
                zbuf[r, 0, pl.ds(kk * 16, 16)] = jnp.zeros((16,), F32)
            return 0

        lax.fori_loop(0, zb, zrow, 0)
        pltpu.sync_copy(tgt_h.at[pl.ds(base, chunk)], idxa)

        def run_pass(read0, read1, drain):
            def zcopy(m_, _):
                roff = tid * nrow + m_ * zb
                pltpu.sync_copy(zbuf, acc.at[pl.ds(roff, zb)])
                return 0

            lax.fori_loop(0, nz, zcopy, 0)
            plsc.subcore_barrier()

            def pair(j2, _):
                o0 = 2 * j2 * eb
                o1 = o0 + eb
                c0 = read0(pl.multiple_of(base + o0, 8), dbuf0, sem0)
                c1 = read1(pl.multiple_of(base + o1, 8), dbuf1, sem1)
                c0.wait()
                pltpu.sync_copy(dbuf0, acc.at[idxa.at[pl.ds(o0, eb)]],
                                add=True)
                c1.wait()
                pltpu.sync_copy(dbuf1, acc.at[idxa.at[pl.ds(o1, eb)]],
                                add=True)
                return 0

            lax.fori_loop(0, nblk // 2, pair, 0)
            plsc.subcore_barrier()
            drain(tid * nrow)
            plsc.subcore_barrier()

        def msg_pass(d):
            def rd(off, buf, sem):
                return pltpu.async_copy(
                    MSG_h.at[pl.ds(off, eb), pl.ds(d, 1), :], buf, sem)

            def dr(roff):
                pltpu.sync_copy(acc.at[pl.ds(roff, nrow)],
                                A_h.at[pl.ds(roff, nrow), pl.ds(d, 1), :])

            run_pass(rd, rd, dr)

        def cm_pass():
            def rd(off, buf, sem):
                return pltpu.async_copy(CM_h.at[pl.ds(off, eb)], buf, sem)

            def dr(roff):
                pltpu.sync_copy(acc.at[pl.ds(roff, nrow)],
                                CMA_h.at[pl.ds(roff, nrow)])

            run_pass(rd, rd, dr)

        @pl.when(cid == 0)
        def _():
            msg_pass(0)
            msg_pass(1)

        @pl.when(cid == 1)
        def _():
            msg_pass(2)
            cm_pass()

    return k(MSG, CM, tgt)


def _node_mlp_body(nf_ref, *refs):
    (out_ref, coord_ref) = refs[-2:]
    nparts = (len(refs) - 10) // 2
    a_refs = refs[:nparts]
    cma_refs = refs[nparts:2 * nparts]
    (c128_ref, d16_ref, vel16_ref, Ut_ref, Vt_ref, bn1_ref, Wn2t_ref,
     bn2_ref) = refs[2 * nparts:-2]
    bn = nf_ref.shape[0]
    A = a_refs[0][...]
    for a in a_refs[1:]:
        A = A + a[...]
    cma = cma_refs[0][...]
    for c in cma_refs[1:]:
        cma = cma + c[...]
    Xn = nf_ref[...].reshape(bn * 3, 128)
    Xa = A.reshape(bn * 3, 128)
    H = _swish(jnp.dot(Xn, Ut_ref[...], preferred_element_type=F32)
               + jnp.dot(Xa, Vt_ref[...], preferred_element_type=F32)
               + bn1_ref[...])
    out_ref[...] = (jnp.dot(H, Wn2t_ref[...], preferred_element_type=F32)
                    + bn2_ref[...] + Xn).reshape(bn, 3, 128)
    coord_ref[...] = (c128_ref[:, :16] + cma[:, 0, :16] / d16_ref[...]
                      + vel16_ref[...])


def _node_mlp(nf, parts, cmas, c128, d16, vel16, Ut, Vt, bn1, Wn2t, bn2):
    n = nf.shape[0]
    bn = 400
    wspec = lambda s: pl.BlockSpec(s, lambda i: (0,) * len(s))
    return pl.pallas_call(
        _node_mlp_body,
        grid=(n // bn,),
        in_specs=[
            pl.BlockSpec((bn, 3, 128), lambda i: (i, 0, 0)),
        ] + [
            pl.BlockSpec((bn, 3, 128), lambda i: (i, 0, 0))
            for _ in parts
        ] + [
            pl.BlockSpec((bn, 1, 128), lambda i: (i, 0, 0))
            for _ in cmas
        ] + [
            pl.BlockSpec((bn, 128), lambda i: (i, 0)),
            pl.BlockSpec((bn, 16), lambda i: (i, 0)),
            pl.BlockSpec((bn, 16), lambda i: (i, 0)),
            wspec((128, 128)), wspec((128, 128)), wspec((1, 128)),
            wspec((128, 128)), wspec((1, 128)),
        ],
        out_specs=[
            pl.BlockSpec((bn, 3, 128), lambda i: (i, 0, 0)),
            pl.BlockSpec((bn, 16), lambda i: (i, 0)),
        ],
        out_shape=[
            jax.ShapeDtypeStruct((n, 3, 128), F32),
            jax.ShapeDtypeStruct((n, 16), F32),
        ],
    )(nf, *parts, *cmas, c128, d16, vel16, Ut, Vt, bn1, Wn2t, bn2)


def kernel(node_feat, degree, coordinate, edge_index, velocity_vector,
           We1, be1, We2, be2, Wc1, bc1, Wc2, bc2,
           Wn1, bn1, Wn2, bn2, Wv1, bv1, Wv2, bv2):
    n = node_feat.shape[0]
    e = edge_index.shape[1]

    At = We1[:, :128].T
    Bt = We1[:, 128:256].T
    w1 = We1[:, 256].reshape(1, 128)
    be1r = be1.reshape(1, 128)
    We2t = We2.T
    be2r = be2.reshape(1, 128)
    Wc1t = Wc1.T
    bc1r = bc1.reshape(1, 128)
    bc2r = bc2.reshape(1, 1)
    ones128 = jnp.ones((128, 128), F32)
    eye3 = jnp.concatenate(
        [jnp.eye(3, dtype=F32), jnp.zeros((3, 125), F32)], axis=1)
    Ws = (Wc2.reshape(1, 128)[:, :, None] * eye3[:, None, :]).reshape(384,
                                                                      128)
    Ut = Wn1[:, :128].T
    Vt = Wn1[:, 128:].T
    bn1r = bn1.reshape(1, 128)
    Wn2t = Wn2.T
    bn2r = bn2.reshape(1, 128)
    Wv1t = Wv1.T
    bv1r = bv1.reshape(1, 128)
    wv2 = Wv2.reshape(1, 128)
    bv2r = bv2.reshape(1, 1)

    v16 = jnp.pad(velocity_vector, ((0, 0), (0, 13)))
    c128 = jnp.pad(coordinate, ((0, 0), (0, 125)))
    d16 = jnp.broadcast_to(degree[:, None], (n, 16))

    P3, Q3, vel16 = _node_pre(node_feat, v16, At, Bt, be1r, Wv1t, bv1r,
                              wv2, bv2r)

    src = edge_index[0]
    tgt = edge_index[1]
    qc = 16 * 80 * 2
    eg = ((e + qc - 1) // qc) * qc
    tgt_g = jnp.pad(tgt, (0, eg - e))
    src_g = jnp.pad(src, (0, eg - e))
    sizes = [10240, 56320, 53760, 30720, 10240]
    assert sum(sizes) == eg and all(s % qc == 0 for s in sizes)

    parts = []
    cmas = []
    off = 0
    for cs_k in sizes:
        tgt_k = lax.slice(tgt_g, (off,), (off + cs_k,))
        src_k = lax.slice(src_g, (off,), (off + cs_k,))
        GTF, GSF, CT, CS = _sc_gather(P3, Q3, c128, tgt_k, src_k)
        MSG, CM = _edge_mlp(GTF, GSF, CT, CS, w1, We2t, be2r, Wc1t, bc1r,
                            ones128, Ws, bc2r, max(0, min(cs_k, e - off)))
        A_k, CMA_k = _sc_scatter(MSG, CM, tgt_k, n)
        parts.append(A_k)
        cmas.append(CMA_k)
        off += cs_k

    new_nf, coord16 = _node_mlp(node_feat, parts, cmas, c128, d16, vel16,
                                Ut, Vt, bn1r, Wn2t, bn2r)

    vel = vel16[:, :3]
    coord = coord16[:, :3]
    return coord, new_nf, vel


# speedup vs baseline: 1.0138x; 1.0138x over previous
import functools

import jax
import jax.numpy as jnp
from jax import lax
from jax.experimental import pallas as pl
from jax.experimental.pallas import tpu as pltpu
from jax.experimental.pallas import tpu_sc as plsc

F32 = jnp.float32


def _swish(x):
    return x * (0.5 + 0.5 * jnp.tanh(0.5 * x))


def _node_pre_body(nf_ref, c128_ref, v16_ref, At_ref, Bt_ref, be1_ref,
                   Wv1t_ref, bv1_ref, wv2_ref, bv2_ref,
                   p_ref, q_ref, vel16_ref):
    bn = nf_ref.shape[0]
    X = nf_ref[...].reshape(bn * 3, 128)
    p_ref[:, :3, :] = (jnp.dot(X, At_ref[...], preferred_element_type=F32)
                       + be1_ref[...]).reshape(bn, 3, 128)
    p_ref[:, 3, :] = c128_ref[...]
    q_ref[:, :3, :] = jnp.dot(X, Bt_ref[...],
                              preferred_element_type=F32).reshape(bn, 3, 128)
    q_ref[:, 3, :] = c128_ref[...]
    V = _swish(jnp.dot(X, Wv1t_ref[...], preferred_element_type=F32)
               + bv1_ref[...])
    sv = jnp.sum(V * wv2_ref[...], axis=1).reshape(bn, 3) + bv2_ref[0, 0]
    sv16 = jnp.concatenate([sv, jnp.zeros((bn, 13), F32)], axis=1)
    vel16_ref[...] = v16_ref[...] * sv16


def _node_pre(nf, c128, v16, At, Bt, be1, Wv1t, bv1, wv2, bv2):
    n = nf.shape[0]
    bn = 400
    wspec = lambda s: pl.BlockSpec(s, lambda i: (0,) * len(s))
    return pl.pallas_call(
        _node_pre_body,
        grid=(n // bn,),
        in_specs=[
            pl.BlockSpec((bn, 3, 128), lambda i: (i, 0, 0)),
            pl.BlockSpec((bn, 128), lambda i: (i, 0)),
            pl.BlockSpec((bn, 16), lambda i: (i, 0)),
            wspec((128, 128)), wspec((128, 128)), wspec((1, 128)),
            wspec((128, 128)), wspec((1, 128)), wspec((1, 128)),
            wspec((1, 1)),
        ],
        out_specs=[
            pl.BlockSpec((bn, 4, 128), lambda i: (i, 0, 0)),
            pl.BlockSpec((bn, 4, 128), lambda i: (i, 0, 0)),
            pl.BlockSpec((bn, 16), lambda i: (i, 0)),
        ],
        out_shape=[
            jax.ShapeDtypeStruct((n, 4, 128), F32),
            jax.ShapeDtypeStruct((n, 4, 128), F32),
            jax.ShapeDtypeStruct((n, 16), F32),
        ],
    )(nf, c128, v16, At, Bt, be1, Wv1t, bv1, wv2, bv2)


def _sc_gather(Pv, Qv, tgt_g, src_g):
    eg = tgt_g.shape[0]
    chunk = eg // 16
    eb = 80
    nblk = chunk // eb
    mesh = plsc.VectorSubcoreMesh(core_axis_name="c", subcore_axis_name="s")

    @functools.partial(
        pl.kernel,
        out_type=[
            jax.ShapeDtypeStruct((eg, 4, 128), F32),
            jax.ShapeDtypeStruct((eg, 4, 128), F32),
        ],
        mesh=mesh,
        scratch_types=[
            pltpu.VMEM((chunk,), jnp.int32),
            pltpu.VMEM((eb, 4, 128), F32),
            pltpu.VMEM((eb, 4, 128), F32),
            pltpu.SemaphoreType.DMA,
            pltpu.SemaphoreType.DMA,
        ],
    )
    def k(P_h, Q_h, tgt_h, src_h, gt_h, gs_h, idxa, buf0, buf1, sem0, sem1):
        cid = lax.axis_index("c")
        tid = lax.axis_index("s")
        base = tid * chunk

        def make(body_src, out_h, idx_h):
            def _():
                pltpu.sync_copy(idx_h.at[pl.ds(base, chunk)], idxa)

                def pair(j2, _):
                    o0 = 2 * j2 * eb
                    o1 = o0 + eb
                    c0 = pltpu.async_copy(
                        body_src.at[idxa.at[pl.ds(o0, eb)]], buf0, sem0)
                    c1 = pltpu.async_copy(
                        body_src.at[idxa.at[pl.ds(o1, eb)]], buf1, sem1)
                    c0.wait()
                    pltpu.sync_copy(
                        buf0, out_h.at[pl.ds(pl.multiple_of(base + o0, 8),
                                             eb)])
                    c1.wait()
                    pltpu.sync_copy(
                        buf1, out_h.at[pl.ds(pl.multiple_of(base + o1, 8),
                                             eb)])
                    return 0

                lax.fori_loop(0, nblk // 2, pair, 0)

            return _

        pl.when(cid == 0)(make(P_h, gt_h, tgt_h))
        pl.when(cid == 1)(make(Q_h, gs_h, src_h))

    return k(Pv, Qv, tgt_g, src_g)


def _edge_mlp_body(gt_ref, gs_ref, w1_ref, We2t_ref, be2_ref,
                   Wc1t_ref, bc1_ref, ones_ref, Ws_ref, bc2_ref, out_ref, *,
                   valid_rows):
    be = gt_ref.shape[0]
    gt = gt_ref[...]
    gs = gs_ref[...]
    relc = gt[:, 3, :] - gs[:, 3, :]
    distb = jnp.dot(relc * relc, ones_ref[...], preferred_element_type=F32)
    X0 = gt[:, :3, :] + gs[:, :3, :] + distb[:, None, :] * w1_ref[...][None]
    X = _swish(X0).reshape(be * 3, 128)
    Y = _swish(jnp.dot(X, We2t_ref[...], preferred_element_type=F32)
               + be2_ref[...])
    m = jnp.where(pl.program_id(0) * be < valid_rows, 1.0, 0.0).astype(F32)
    out_ref[:, :3, :] = Y.reshape(be, 3, 128) * m
    C = _swish(jnp.dot(Y, Wc1t_ref[...], preferred_element_type=F32)
               + bc1_ref[...]).reshape(be, 3, 128)
    s128 = (jnp.dot(C[:, 0, :], Ws_ref[0:128, :],
                    preferred_element_type=F32)
            + jnp.dot(C[:, 1, :], Ws_ref[128:256, :],
                      preferred_element_type=F32)
            + jnp.dot(C[:, 2, :], Ws_ref[256:384, :],
                      preferred_element_type=F32))
    out_ref[:, 3, :] = relc * (s128 + bc2_ref[0, 0]) * m


def _edge_mlp(gt, gs, w1, We2t, be2, Wc1t, bc1, ones128, Ws, bc2, valid_rows):
    eg = gt.shape[0]
    be = 640
    wspec = lambda s: pl.BlockSpec(s, lambda i: (0,) * len(s))
    return pl.pallas_call(
        functools.partial(_edge_mlp_body, valid_rows=valid_rows),
        grid=(eg // be,),
        in_specs=[
            pl.BlockSpec((be, 4, 128), lambda i: (i, 0, 0)),
            pl.BlockSpec((be, 4, 128), lambda i: (i, 0, 0)),
            wspec((1, 128)), wspec((128, 128)), wspec((1, 128)),
            wspec((128, 128)), wspec((1, 128)), wspec((128, 128)),
            wspec((384, 128)), wspec((1, 1)),
        ],
        out_specs=pl.BlockSpec((be, 4, 128), lambda i: (i, 0, 0)),
        out_shape=jax.ShapeDtypeStruct((eg, 4, 128), F32),
    )(gt, gs, w1, We2t, be2, Wc1t, bc1, ones128, Ws, bc2)


def _sc_scatter(E3, tgt, n):
    e = tgt.shape[0]
    nsub = 16
    chunk = e // nsub
    eb = 80
    nblk = chunk // eb
    nrow = n // nsub
    zb = 25
    nz = nrow // zb
    mesh = plsc.VectorSubcoreMesh(core_axis_name="c", subcore_axis_name="s")

    @functools.partial(
        pl.kernel,
        out_type=jax.ShapeDtypeStruct((n, 4, 128), F32),
        mesh=mesh,
        scratch_types=[
            pltpu.VMEM_SHARED((n, 1, 128), F32),
            pltpu.VMEM((chunk,), jnp.int32),
            pltpu.VMEM((eb, 1, 128), F32),
            pltpu.VMEM((eb, 1, 128), F32),
            pltpu.VMEM((zb, 1, 128), F32),
            pltpu.SemaphoreType.DMA,
            pltpu.SemaphoreType.DMA,
        ],
    )
    def k(E_h, tgt_h, A_h, acc, idxa, dbuf0, dbuf1, zbuf, sem0, sem1):
        cid = lax.axis_index("c")
        tid = lax.axis_index("s")
        base = tid * chunk

        def zrow(r, _):
            for kk in range(8):
                zbuf[r, 0, pl.ds(kk * 16, 16)] = jnp.zeros((16,), F32)
            return 0

        lax.fori_loop(0, zb, zrow, 0)
        pltpu.sync_copy(tgt_h.at[pl.ds(base, chunk)], idxa)

        for p in range(2):
            kchunk = 2 * cid + p

            def zcopy(m, _):
                roff = tid * nrow + m * zb
                pltpu.sync_copy(zbuf, acc.at[pl.ds(roff, zb)])
                return 0

            lax.fori_loop(0, nz, zcopy, 0)
            plsc.subcore_barrier()

            def pair(j2, _):
                o0 = 2 * j2 * eb
                o1 = o0 + eb
                c0 = pltpu.async_copy(
                    E_h.at[pl.ds(pl.multiple_of(base + o0, 8), eb),
                           pl.ds(kchunk, 1), :], dbuf0, sem0)
                c1 = pltpu.async_copy(
                    E_h.at[pl.ds(pl.multiple_of(base + o1, 8), eb),
                           pl.ds(kchunk, 1), :], dbuf1, sem1)
                c0.wait()
                pltpu.sync_copy(dbuf0, acc.at[idxa.at[pl.ds(o0, eb)]],
                                add=True)
                c1.wait()
                pltpu.sync_copy(dbuf1, acc.at[idxa.at[pl.ds(o1, eb)]],
                                add=True)
                return 0

            lax.fori_loop(0, nblk // 2, pair, 0)
            plsc.subcore_barrier()

            roff = tid * nrow
            pltpu.sync_copy(acc.at[pl.ds(roff, nrow)],
                            A_h.at[pl.ds(roff, nrow), pl.ds(kchunk, 1), :])
            plsc.subcore_barrier()

    return k(E3, tgt)


def _node_mlp_body(nf_ref, *refs):
    (out_ref, coord_ref) = refs[-2:]
    nparts = len(refs) - 10
    a_refs = refs[:nparts]
    (c128_ref, d16_ref, vel16_ref, Ut_ref, Vt_ref, bn1_ref, Wn2t_ref,
     bn2_ref) = refs[nparts:-2]
    bn = nf_ref.shape[0]
    A = a_refs[0][...]
    for a in a_refs[1:]:
        A = A + a[...]
    Xn = nf_ref[...].reshape(bn * 3, 128)
    Xa = A[:, :3, :].reshape(bn * 3, 128)
    H = _swish(jnp.dot(Xn, Ut_ref[...], preferred_element_type=F32)
               + jnp.dot(Xa, Vt_ref[...], preferred_element_type=F32)
               + bn1_ref[...])
    out_ref[...] = (jnp.dot(H, Wn2t_ref[...], preferred_element_type=F32)
                    + bn2_ref[...] + Xn).reshape(bn, 3, 128)
    cm16 = A[:, 3, :16]
    coord_ref[...] = (c128_ref[:, :16] + cm16 / d16_ref[...]
                      + vel16_ref[...])


def _node_mlp(nf, parts, c128, d16, vel16, Ut, Vt, bn1, Wn2t, bn2):
    n = nf.shape[0]
    bn = 400
    wspec = lambda s: pl.BlockSpec(s, lambda i: (0,) * len(s))
    return pl.pallas_call(
        _node_mlp_body,
        grid=(n // bn,),
        in_specs=[
            pl.BlockSpec((bn, 3, 128), lambda i: (i, 0, 0)),
        ] + [
            pl.BlockSpec((bn, 4, 128), lambda i: (i, 0, 0))
            for _ in parts
        ] + [
            pl.BlockSpec((bn, 128), lambda i: (i, 0)),
            pl.BlockSpec((bn, 16), lambda i: (i, 0)),
            pl.BlockSpec((bn, 16), lambda i: (i, 0)),
            wspec((128, 128)), wspec((128, 128)), wspec((1, 128)),
            wspec((128, 128)), wspec((1, 128)),
        ],
        out_specs=[
            pl.BlockSpec((bn, 3, 128), lambda i: (i, 0, 0)),
            pl.BlockSpec((bn, 16), lambda i: (i, 0)),
        ],
        out_shape=[
            jax.ShapeDtypeStruct((n, 3, 128), F32),
            jax.ShapeDtypeStruct((n, 16), F32),
        ],
    )(nf, *parts, c128, d16, vel16, Ut, Vt, bn1, Wn2t, bn2)


def kernel(node_feat, degree, coordinate, edge_index, velocity_vector,
           We1, be1, We2, be2, Wc1, bc1, Wc2, bc2,
           Wn1, bn1, Wn2, bn2, Wv1, bv1, Wv2, bv2):
    n = node_feat.shape[0]
    e = edge_index.shape[1]

    At = We1[:, :128].T
    Bt = We1[:, 128:256].T
    w1 = We1[:, 256].reshape(1, 128)
    be1r = be1.reshape(1, 128)
    We2t = We2.T
    be2r = be2.reshape(1, 128)
    Wc1t = Wc1.T
    bc1r = bc1.reshape(1, 128)
    bc2r = bc2.reshape(1, 1)
    ones128 = jnp.ones((128, 128), F32)
    eye3 = jnp.concatenate(
        [jnp.eye(3, dtype=F32), jnp.zeros((3, 125), F32)], axis=1)
    Ws = (Wc2.reshape(1, 128)[:, :, None] * eye3[:, None, :]).reshape(384,
                                                                      128)
    Ut = Wn1[:, :128].T
    Vt = Wn1[:, 128:].T
    bn1r = bn1.reshape(1, 128)
    Wn2t = Wn2.T
    bn2r = bn2.reshape(1, 128)
    Wv1t = Wv1.T
    bv1r = bv1.reshape(1, 128)
    wv2 = Wv2.reshape(1, 128)
    bv2r = bv2.reshape(1, 1)

    v16 = jnp.pad(velocity_vector, ((0, 0), (0, 13)))
    c128 = jnp.pad(coordinate, ((0, 0), (0, 125)))
    d16 = jnp.broadcast_to(degree[:, None], (n, 16))

    P, Q, vel16 = _node_pre(node_feat, c128, v16, At, Bt, be1r, Wv1t, bv1r,
                            wv2, bv2r)

    src = edge_index[0]
    tgt = edge_index[1]
    qc = 16 * 80 * 2
    eg = ((e + qc - 1) // qc) * qc
    tgt_g = jnp.pad(tgt, (0, eg - e))
    src_g = jnp.pad(src, (0, eg - e))
    sizes = [10240, 56320, 53760, 30720, 10240]
    assert sum(sizes) == eg and all(s % qc == 0 for s in sizes)

    parts = []
    off = 0
    for cs_k in sizes:
        tgt_k = lax.slice(tgt_g, (off,), (off + cs_k,))
        src_k = lax.slice(src_g, (off,), (off + cs_k,))
        GT, GS = _sc_gather(P, Q, tgt_k, src_k)
        E = _edge_mlp(GT, GS, w1, We2t, be2r, Wc1t, bc1r, ones128, Ws, bc2r,
                      max(0, min(cs_k, e - off)))
        parts.append(_sc_scatter(E, tgt_k, n))
        off += cs_k

    new_nf, coord16 = _node_mlp(node_feat, parts, c128, d16, vel16, Ut, Vt,
                                bn1r, Wn2t, bn2r)

    vel = vel16[:, :3]
    coord = coord16[:, :3]
    return coord, new_nf, vel
